# baseline (speedup 1.0000x reference)
import functools

import jax
import jax.numpy as jnp
from jax import lax
from jax.experimental import pallas as pl
from jax.experimental.pallas import tpu as pltpu
from jax.experimental.pallas import tpu_sc as plsc

F32 = jnp.float32
I32 = jnp.int32

N = 10000
E = 160000
EMB = 300
NLAYER = 5
G = 256

NP = 10240
BM = 512
NB = NP // BM
F = 320
FH = 160
H2 = 640
PD = 256

NC, NS = 2, 16
NPT = NP // NS
CH = 128

@functools.cache
def _sc_mesh():
    return plsc.VectorSubcoreMesh(
        core_axis_name="c", subcore_axis_name="s",
        num_cores=NC, num_subcores=NS)


EPW = E // (NC * NS)
NCH_W = EPW // CH
TW = EPW - NCH_W * CH


def _counts_body(pk2, z16, out, pk_all, dst_v, oh_v, dst_t, oh_t, cnt_sh, pks, scs):
    c = lax.axis_index("c")
    s = lax.axis_index("s")
    w = c * NS + s
    sl = pl.ds(s * NPT, NPT)
    pltpu.sync_copy(z16, cnt_sh.at[sl])
    base0 = w * EPW
    pltpu.async_copy(pk2.at[pl.ds(base0, EPW)], pk_all, pks).wait()
    plsc.subcore_barrier()

    ones16 = jnp.ones((16,), F32)
    zeros16 = jnp.zeros((16,), F32)

    def decode(m, ng, mask):
        idxs = []
        for g in range(ng):
            jj = lax.iota(I32, 16) + g * 16
            if mask is not None:
                p = plsc.load_gather(pk_all, [jnp.where(mask, m * CH + jj, 0)])
                jjs = jnp.where(mask, jj, 0)
                dst_t[...] = lax.shift_right_logical(p, 16)
            else:
                p = pk_all[pl.ds(m * CH + g * 16, 16)]
                jjs = jj
                dst_v[pl.ds(g * 16, 16)] = lax.shift_right_logical(p, 16)
            a0 = lax.shift_right_logical(p, 8) & 0xFF
            a1 = p & 0xFF
            cc = a0 * 3 + a1
            plsc.store_scatter(oh_t if mask is not None else oh_v,
                               [jjs, cc], ones16, mask=mask)
            idxs.append((jjs, cc))
        return idxs

    def body(i, carry):
        idxs = decode(i, CH // 16, None)
        pltpu.async_copy(oh_v, cnt_sh.at[dst_v], scs, add=True).wait()
        for jjs, cc in idxs:
            plsc.store_scatter(oh_v, [jjs, cc], zeros16)
        return carry

    lax.fori_loop(0, NCH_W, body, 0)

    tm = lax.iota(I32, 16) < TW
    idxs = decode(NCH_W, 1, tm)
    pltpu.async_copy(oh_t, cnt_sh.at[dst_t], scs, add=True).wait()
    for jjs, cc in idxs:
        plsc.store_scatter(oh_t, [jjs, cc], zeros16, mask=tm)

    plsc.subcore_barrier()
    pltpu.sync_copy(cnt_sh.at[sl], out.at[c, sl])


@functools.cache
def _counts_kernel():
    return pl.kernel(
        _counts_body,
        out_type=jax.ShapeDtypeStruct((NC, NP, 16), F32),
        mesh=_sc_mesh(),
        scratch_types=[
            pltpu.VMEM((EPW,), I32),
            pltpu.VMEM((CH,), I32),
            pltpu.VMEM((CH, 16), F32),
            pltpu.VMEM((16,), I32),
            pltpu.VMEM((16, 16), F32),
            pltpu.VMEM_SHARED((NP, 16), F32),
            pltpu.SemaphoreType.DMA,
            pltpu.SemaphoreType.DMA,
        ],
        compiler_params=pltpu.CompilerParams(use_tc_tiling_on_sc=False, needs_layout_passes=False),
    )


def _counts_call(pk2, z16):
    return _counts_kernel()(pk2, z16)


EPT = E // NS
CHS = 80
NCHS = EPT // CHS
NPS = N
NPTS = NPS // NS


def _spmm_body(hL, hR, sd3, zrow, aggL, aggR,
               sd0, sd1, sd2, sd3b, rows0, rows1, agg_sh,
               qs0, qs1, qs2, qs3, gg0, gg1, scat):
    c = lax.axis_index("c")
    s = lax.axis_index("s")
    sl = pl.ds(s * NPTS, NPTS)
    pltpu.sync_copy(zrow, agg_sh.at[sl])
    plsc.subcore_barrier()

    rbase = s * NCHS
    sd_b = (sd0, sd1, sd2, sd3b)
    rows_b = (rows0, rows1)
    qsem = (qs0, qs1, qs2, qs3)
    gsem = (gg0, gg1)
    LAST = NCHS - 1

    def do_edges(h_hbm):
        def start_idx(i, k4):
            pltpu.async_copy(sd3.at[rbase + i], sd_b[k4], qsem[k4])

        def step(i, k4, k41, k42, r, r1):
            @pl.when(i < LAST)
            def _():
                pltpu.make_async_copy(sd3.at[rbase + i + 1], sd_b[k41],
                                      qsem[k41]).wait()
            @pl.when(i > 0)
            def _():
                pltpu.make_async_copy(rows_b[r1], agg_sh.at[sd_b[k41].at[1]],
                                      scat).wait()
            @pl.when(i < LAST)
            def _():
                pltpu.async_copy(h_hbm.at[sd_b[k41].at[0]], rows_b[r1],
                                 gsem[r1])
            @pl.when(i + 1 < LAST)
            def _():
                start_idx(i + 2, k42)
            pltpu.make_async_copy(h_hbm.at[sd_b[k4].at[0]], rows_b[r],
                                  gsem[r]).wait()
            pltpu.async_copy(rows_b[r], agg_sh.at[sd_b[k4].at[1]], scat,
                             add=True)

        start_idx(0, 0)
        start_idx(1, 1)
        pltpu.make_async_copy(sd3.at[rbase], sd_b[0], qsem[0]).wait()
        pltpu.async_copy(h_hbm.at[sd_b[0].at[0]], rows_b[0], gsem[0])

        def group(j, carry):
            i0 = j * 4
            step(i0, 0, 1, 2, 0, 1)
            step(i0 + 1, 1, 2, 3, 1, 0)
            step(i0 + 2, 2, 3, 0, 0, 1)
            step(i0 + 3, 3, 0, 1, 1, 0)
            return carry
        lax.fori_loop(0, NCHS // 4, group, 0)
        step(LAST, 0, 1, 2, 0, 1)
        pltpu.make_async_copy(rows_b[0], agg_sh.at[sd_b[0].at[1]],
                              scat).wait()

    @pl.when(c == 0)
    def _():
        do_edges(hL)

    @pl.when(c == 1)
    def _():
        do_edges(hR)

    plsc.subcore_barrier()

    @pl.when(c == 0)
    def _():
        pltpu.sync_copy(agg_sh.at[sl], aggL.at[pl.ds(s * NPTS, NPTS)])

    @pl.when(c == 1)
    def _():
        pltpu.sync_copy(agg_sh.at[sl], aggR.at[pl.ds(s * NPTS, NPTS)])


@functools.cache
def _spmm_kernel():
    return pl.kernel(
        _spmm_body,
        out_type=(jax.ShapeDtypeStruct((NP, FH), F32),
                  jax.ShapeDtypeStruct((NP, FH), F32)),
        mesh=_sc_mesh(),
        scratch_types=[
            pltpu.VMEM((2, CHS), I32),
            pltpu.VMEM((2, CHS), I32),
            pltpu.VMEM((2, CHS), I32),
            pltpu.VMEM((2, CHS), I32),
            pltpu.VMEM((CHS, FH), F32),
            pltpu.VMEM((CHS, FH), F32),
            pltpu.VMEM_SHARED((NPS, FH), F32),
        ] + [pltpu.SemaphoreType.DMA] * 7,
        compiler_params=pltpu.CompilerParams(use_tc_tiling_on_sc=False, needs_layout_passes=False),
    )


def _spmm_call(hL, hR, sd3, zrow):
    return _spmm_kernel()(hL, hR, sd3, zrow)


def _embed_body(x_ref, tab_ref, oL_ref, oR_ref):
    x = x_ref[0]
    xc = x[:, 0:1] * 3 + x[:, 1:2]
    io = lax.broadcasted_iota(I32, (1, 16), 1)
    oh = (xc == io).astype(F32)
    h0 = jnp.dot(oh, tab_ref[...], preferred_element_type=F32)
    oL_ref[...] = h0[:, :FH]
    oR_ref[...] = h0[:, FH:]


def _embed_call(x0r, tab16):
    return pl.pallas_call(
        _embed_body,
        grid=(NB,),
        in_specs=[
            pl.BlockSpec((1, BM, 2), lambda i: (i, 0, 0)),
            pl.BlockSpec((16, F), lambda i: (0, 0)),
        ],
        out_specs=(pl.BlockSpec((BM, FH), lambda i: (i, 0)),
                   pl.BlockSpec((BM, FH), lambda i: (i, 0))),
        out_shape=(jax.ShapeDtypeStruct((NP, FH), F32),
                   jax.ShapeDtypeStruct((NP, FH), F32)),
    )(x0r, tab16)


def _mlp_body(relu_out, aL, aR, hL, hR, cn, et, se, w1, b1, w2, b2, gm, bt,
              oL_ref, oR_ref):
    agg = jnp.concatenate([aL[...], aR[...]], axis=1)
    agg = agg + jnp.concatenate([hL[...], hR[...]], axis=1)
    cnt = cn[0] + cn[1]
    agg = agg + se[...] + jnp.dot(cnt, et[...], preferred_element_type=F32)
    t = jnp.maximum(jnp.dot(agg, w1[...], preferred_element_type=F32) + b1[...], 0.0)
    h = jnp.dot(t, w2[...], preferred_element_type=F32) + b2[...]
    h = h * gm[...] + bt[...]
    if relu_out:
        h = jnp.maximum(h, 0.0)
    oL_ref[...] = h[:, :FH]
    oR_ref[...] = h[:, FH:]


def _mlp_call(relu_out, aL, aR, hL, hR, cnt2, et, se, w1, b1, w2, b2, gm, bt):
    half = pl.BlockSpec((BM, FH), lambda i: (i, 0))
    return pl.pallas_call(
        functools.partial(_mlp_body, relu_out),
        grid=(NB,),
        in_specs=[
            half, half, half, half,
            pl.BlockSpec((NC, BM, 16), lambda i: (0, i, 0)),
            pl.BlockSpec((16, F), lambda i: (0, 0)),
            pl.BlockSpec((1, F), lambda i: (0, 0)),
            pl.BlockSpec((F, H2), lambda i: (0, 0)),
            pl.BlockSpec((1, H2), lambda i: (0, 0)),
            pl.BlockSpec((H2, F), lambda i: (0, 0)),
            pl.BlockSpec((1, F), lambda i: (0, 0)),
            pl.BlockSpec((1, F), lambda i: (0, 0)),
            pl.BlockSpec((1, F), lambda i: (0, 0)),
        ],
        out_specs=(half, half),
        out_shape=(jax.ShapeDtypeStruct((NP, FH), F32),
                   jax.ShapeDtypeStruct((NP, FH), F32)),
    )(aL, aR, hL, hR, cnt2, et, se, w1, b1, w2, b2, gm, bt)


def _pool_body(hL, hR, b_ref, p1, pb1, p2, pb2, out_ref):
    i = pl.program_id(0)
    h = jnp.concatenate([hL[...], hR[...]], axis=1)
    t = jnp.maximum(jnp.dot(h, p1[...], preferred_element_type=F32) + pb1[...], 0.0)
    x = jnp.dot(t, p2[...], preferred_element_type=F32) + pb2[...]
    lane = lax.broadcasted_iota(I32, (1, PD), 1)
    x = x + (lane == PD - 1).astype(F32)
    b = b_ref[0]
    gio = lax.broadcasted_iota(I32, (G, BM), 0)
    ohT = (gio == b).astype(F32)
    contrib = jnp.dot(ohT, x, preferred_element_type=F32)

    @pl.when(i == 0)
    def _():
        out_ref[...] = contrib

    @pl.when(i > 0)
    def _():
        out_ref[...] = out_ref[...] + contrib

    @pl.when(i == NB - 1)
    def _():
        sums = out_ref[...]
        cnt = jnp.maximum(sums[:, PD - 1:PD], 1.0)
        out_ref[...] = sums / cnt


def _pool_call(hL, hR, batchr, p1, pb1, p2, pb2):
    half = pl.BlockSpec((BM, FH), lambda i: (i, 0))
    return pl.pallas_call(
        _pool_body,
        grid=(NB,),
        in_specs=[
            half, half,
            pl.BlockSpec((1, 1, BM), lambda i: (i, 0, 0)),
            pl.BlockSpec((F, F), lambda i: (0, 0)),
            pl.BlockSpec((1, F), lambda i: (0, 0)),
            pl.BlockSpec((F, PD), lambda i: (0, 0)),
            pl.BlockSpec((1, PD), lambda i: (0, 0)),
        ],
        out_specs=pl.BlockSpec((G, PD), lambda i: (0, 0)),
        out_shape=jax.ShapeDtypeStruct((G, PD), F32),
    )(hL, hR, batchr, p1, pb1, p2, pb2)


def kernel(x0, edge_index0, edge_attr, batch, atom_emb1, atom_emb2,
           edge_emb1, edge_emb2, W1, b1, W2, b2, gamma, beta, P1, pb1, P2, pb2):
    inv_std = 1.0 / jnp.sqrt(jnp.asarray(1.0 + 1e-5, F32))

    def padf(a, rows, cols):
        return jnp.zeros((rows, cols), F32).at[:a.shape[0], :a.shape[1]].set(a)

    t9 = (atom_emb1[:3][:, None, :] + atom_emb2[None, :3, :]).reshape(9, EMB)
    tab16 = padf(t9, 16, F)
    e9 = (edge_emb1[:, :3][:, :, None, :]
          + edge_emb2[:, None, :3, :]).reshape(NLAYER, 9, EMB)
    etabs = [padf(e9[l], 16, F) for l in range(NLAYER)]
    selfs = [padf((edge_emb1[l, 4] + edge_emb2[l, 0])[None, :], 1, F)
             for l in range(NLAYER)]
    W1p = [padf(W1[l], F, H2) for l in range(NLAYER)]
    b1p = [padf(b1[l][None, :], 1, H2) for l in range(NLAYER)]
    W2p = [padf(W2[l], H2, F) for l in range(NLAYER)]
    b2p = [padf(b2[l][None, :], 1, F) for l in range(NLAYER)]
    gmp = [padf((gamma[l] * inv_std)[None, :], 1, F) for l in range(NLAYER)]
    btp = [padf(beta[l][None, :], 1, F) for l in range(NLAYER)]
    P1p = padf(P1, F, F)
    pb1p = padf(pb1[None, :], 1, F)
    P2p = padf(P2, F, PD)
    pb2p = padf(pb2[None, :], 1, PD)

    x0r = jnp.zeros((NP, 2), I32).at[:N].set(x0.astype(I32)).reshape(NB, BM, 2)
    batchr = jnp.full((NP,), -1, I32).at[:N].set(batch.astype(I32)).reshape(NB, 1, BM)
    srca = edge_index0[0].astype(I32)
    dsta = edge_index0[1].astype(I32)
    ea = edge_attr.astype(I32)
    pk2 = (dsta << 16) | (ea[:, 0] << 8) | ea[:, 1]
    sd3a = jnp.stack([srca.reshape(E // CHS, CHS),
                      dsta.reshape(E // CHS, CHS)], axis=1)
    z16 = jnp.zeros((NPT, 16), F32)
    zrow = jnp.zeros((NPTS, FH), F32)

    hL, hR = _embed_call(x0r, tab16)
    cnt2 = _counts_call(pk2, z16)

    for l in range(NLAYER):
        aggL, aggR = _spmm_call(hL, hR, sd3a, zrow)
        hL, hR = _mlp_call(l < NLAYER - 1, aggL, aggR, hL, hR, cnt2,
                           etabs[l], selfs[l], W1p[l], b1p[l], W2p[l], b2p[l],
                           gmp[l], btp[l])

    pooled = _pool_call(hL, hR, batchr, P1p, pb1p, P2p, pb2p)
    return pooled[:, :150].reshape(-1)


# speedup vs baseline: 1.0078x; 1.0078x over previous
import functools

import jax
import jax.numpy as jnp
from jax import lax
from jax.experimental import pallas as pl
from jax.experimental.pallas import tpu as pltpu
from jax.experimental.pallas import tpu_sc as plsc

F32 = jnp.float32
I32 = jnp.int32

N = 10000
E = 160000
EMB = 300
NLAYER = 5
G = 256

NP = 10240
BM = 512
NB = NP // BM
F = 320
FH = 160
H2 = 640
PD = 256

NC, NS = 2, 16
NPT = NP // NS
CH = 128

@functools.cache
def _sc_mesh():
    return plsc.VectorSubcoreMesh(
        core_axis_name="c", subcore_axis_name="s",
        num_cores=NC, num_subcores=NS)


EPW = E // (NC * NS)
NCH_W = EPW // CH
TW = EPW - NCH_W * CH


def _counts_body(pk2, z16, out, pk_all, dst_v, oh_v, dst_t, oh_t, cnt_sh, pks, scs):
    c = lax.axis_index("c")
    s = lax.axis_index("s")
    w = c * NS + s
    sl = pl.ds(s * NPT, NPT)
    pltpu.sync_copy(z16, cnt_sh.at[sl])
    base0 = w * EPW
    pltpu.async_copy(pk2.at[pl.ds(base0, EPW)], pk_all, pks).wait()
    plsc.subcore_barrier()

    ones16 = jnp.ones((16,), F32)
    zeros16 = jnp.zeros((16,), F32)

    def decode(m, ng, mask):
        idxs = []
        for g in range(ng):
            jj = lax.iota(I32, 16) + g * 16
            if mask is not None:
                p = plsc.load_gather(pk_all, [jnp.where(mask, m * CH + jj, 0)])
                jjs = jnp.where(mask, jj, 0)
                dst_t[...] = lax.shift_right_logical(p, 16)
            else:
                p = pk_all[pl.ds(m * CH + g * 16, 16)]
                jjs = jj
                dst_v[pl.ds(g * 16, 16)] = lax.shift_right_logical(p, 16)
            a0 = lax.shift_right_logical(p, 8) & 0xFF
            a1 = p & 0xFF
            cc = a0 * 3 + a1
            plsc.store_scatter(oh_t if mask is not None else oh_v,
                               [jjs, cc], ones16, mask=mask)
            idxs.append((jjs, cc))
        return idxs

    def body(i, carry):
        idxs = decode(i, CH // 16, None)
        pltpu.async_copy(oh_v, cnt_sh.at[dst_v], scs, add=True).wait()
        for jjs, cc in idxs:
            plsc.store_scatter(oh_v, [jjs, cc], zeros16)
        return carry

    lax.fori_loop(0, NCH_W, body, 0)

    tm = lax.iota(I32, 16) < TW
    idxs = decode(NCH_W, 1, tm)
    pltpu.async_copy(oh_t, cnt_sh.at[dst_t], scs, add=True).wait()
    for jjs, cc in idxs:
        plsc.store_scatter(oh_t, [jjs, cc], zeros16, mask=tm)

    plsc.subcore_barrier()
    pltpu.sync_copy(cnt_sh.at[sl], out.at[c, sl])


@functools.cache
def _counts_kernel():
    return pl.kernel(
        _counts_body,
        out_type=jax.ShapeDtypeStruct((NC, NP, 16), F32),
        mesh=_sc_mesh(),
        scratch_types=[
            pltpu.VMEM((EPW,), I32),
            pltpu.VMEM((CH,), I32),
            pltpu.VMEM((CH, 16), F32),
            pltpu.VMEM((16,), I32),
            pltpu.VMEM((16, 16), F32),
            pltpu.VMEM_SHARED((NP, 16), F32),
            pltpu.SemaphoreType.DMA,
            pltpu.SemaphoreType.DMA,
        ],
        compiler_params=pltpu.CompilerParams(use_tc_tiling_on_sc=False, needs_layout_passes=False),
    )


def _counts_call(pk2, z16):
    return _counts_kernel()(pk2, z16)


EPT = E // NS
CHS = 80
NCHS = EPT // CHS
NPS = N
NPTS = NPS // NS


def _spmm_body(hL, hR, srca, dsta, zrow, aggL, aggR,
               src0, src1, src2, src3, dst0, dst1, dst2, dst3,
               rows0, rows1, agg_sh,
               ss0, ss1, ss2, ss3, dd0, dd1, dd2, dd3, gg0, gg1, scat):
    c = lax.axis_index("c")
    s = lax.axis_index("s")
    sl = pl.ds(s * NPTS, NPTS)
    pltpu.sync_copy(zrow, agg_sh.at[sl])
    plsc.subcore_barrier()

    base0 = s * EPT
    src_b = (src0, src1, src2, src3)
    dst_b = (dst0, dst1, dst2, dst3)
    rows_b = (rows0, rows1)
    ssem = (ss0, ss1, ss2, ss3)
    dsem = (dd0, dd1, dd2, dd3)
    gsem = (gg0, gg1)
    LAST = NCHS - 1

    def do_edges(h_hbm):
        def start_idx(i, k4):
            pltpu.async_copy(srca.at[pl.ds(base0 + i * CHS, CHS)],
                             src_b[k4], ssem[k4])
            pltpu.async_copy(dsta.at[pl.ds(base0 + i * CHS, CHS)],
                             dst_b[k4], dsem[k4])

        def step(i, k4, k41, k42, r, r1):
            @pl.when(i < LAST)
            def _():
                pltpu.make_async_copy(srca.at[pl.ds(base0 + (i + 1) * CHS, CHS)],
                                      src_b[k41], ssem[k41]).wait()
            @pl.when(i > 0)
            def _():
                pltpu.make_async_copy(rows_b[r1], agg_sh.at[dst_b[k41]],
                                      scat).wait()
            @pl.when(i < LAST)
            def _():
                pltpu.async_copy(h_hbm.at[src_b[k41]], rows_b[r1], gsem[r1])
            @pl.when(i + 1 < LAST)
            def _():
                start_idx(i + 2, k42)
            pltpu.make_async_copy(h_hbm.at[src_b[k4]], rows_b[r],
                                  gsem[r]).wait()
            pltpu.make_async_copy(dsta.at[pl.ds(base0 + i * CHS, CHS)],
                                  dst_b[k4], dsem[k4]).wait()
            pltpu.async_copy(rows_b[r], agg_sh.at[dst_b[k4]], scat, add=True)

        start_idx(0, 0)
        start_idx(1, 1)
        pltpu.make_async_copy(srca.at[pl.ds(base0, CHS)], src_b[0],
                              ssem[0]).wait()
        pltpu.async_copy(h_hbm.at[src_b[0]], rows_b[0], gsem[0])

        def group(j, carry):
            i0 = j * 4
            step(i0, 0, 1, 2, 0, 1)
            step(i0 + 1, 1, 2, 3, 1, 0)
            step(i0 + 2, 2, 3, 0, 0, 1)
            step(i0 + 3, 3, 0, 1, 1, 0)
            return carry
        lax.fori_loop(0, (NCHS - 1) // 4, group, 0)
        step(LAST, 0, 1, 2, 0, 1)
        pltpu.make_async_copy(rows_b[0], agg_sh.at[dst_b[0]], scat).wait()

    @pl.when(c == 0)
    def _():
        do_edges(hL)

    @pl.when(c == 1)
    def _():
        do_edges(hR)

    plsc.subcore_barrier()

    @pl.when(c == 0)
    def _():
        pltpu.sync_copy(agg_sh.at[sl], aggL.at[pl.ds(s * NPTS, NPTS)])

    @pl.when(c == 1)
    def _():
        pltpu.sync_copy(agg_sh.at[sl], aggR.at[pl.ds(s * NPTS, NPTS)])


@functools.cache
def _spmm_kernel():
    return pl.kernel(
        _spmm_body,
        out_type=(jax.ShapeDtypeStruct((NP, FH), F32),
                  jax.ShapeDtypeStruct((NP, FH), F32)),
        mesh=_sc_mesh(),
        scratch_types=[
            pltpu.VMEM((CHS,), I32),
            pltpu.VMEM((CHS,), I32),
            pltpu.VMEM((CHS,), I32),
            pltpu.VMEM((CHS,), I32),
            pltpu.VMEM((CHS,), I32),
            pltpu.VMEM((CHS,), I32),
            pltpu.VMEM((CHS,), I32),
            pltpu.VMEM((CHS,), I32),
            pltpu.VMEM((CHS, FH), F32),
            pltpu.VMEM((CHS, FH), F32),
            pltpu.VMEM_SHARED((NPS, FH), F32),
        ] + [pltpu.SemaphoreType.DMA] * 11,
        compiler_params=pltpu.CompilerParams(use_tc_tiling_on_sc=False, needs_layout_passes=False),
    )


def _spmm_call(hL, hR, srca, dsta, zrow):
    return _spmm_kernel()(hL, hR, srca, dsta, zrow)


def _embed_body(x_ref, tab_ref, oL_ref, oR_ref):
    x = x_ref[0]
    xc = x[:, 0:1] * 3 + x[:, 1:2]
    io = lax.broadcasted_iota(I32, (1, 16), 1)
    oh = (xc == io).astype(F32)
    h0 = jnp.dot(oh, tab_ref[...], preferred_element_type=F32)
    oL_ref[...] = h0[:, :FH]
    oR_ref[...] = h0[:, FH:]


def _embed_call(x0r, tab16):
    return pl.pallas_call(
        _embed_body,
        grid=(NB,),
        in_specs=[
            pl.BlockSpec((1, BM, 2), lambda i: (i, 0, 0)),
            pl.BlockSpec((16, F), lambda i: (0, 0)),
        ],
        out_specs=(pl.BlockSpec((BM, FH), lambda i: (i, 0)),
                   pl.BlockSpec((BM, FH), lambda i: (i, 0))),
        out_shape=(jax.ShapeDtypeStruct((NP, FH), F32),
                   jax.ShapeDtypeStruct((NP, FH), F32)),
    )(x0r, tab16)


def _mlp_body(relu_out, aL, aR, hL, hR, cn, et, se, w1, b1, w2, b2, gm, bt,
              oL_ref, oR_ref):
    agg = jnp.concatenate([aL[...], aR[...]], axis=1)
    agg = agg + jnp.concatenate([hL[...], hR[...]], axis=1)
    cnt = cn[0] + cn[1]
    agg = agg + se[...] + jnp.dot(cnt, et[...], preferred_element_type=F32)
    t = jnp.maximum(jnp.dot(agg, w1[...], preferred_element_type=F32) + b1[...], 0.0)
    h = jnp.dot(t, w2[...], preferred_element_type=F32) + b2[...]
    h = h * gm[...] + bt[...]
    if relu_out:
        h = jnp.maximum(h, 0.0)
    oL_ref[...] = h[:, :FH]
    oR_ref[...] = h[:, FH:]


def _mlp_call(relu_out, aL, aR, hL, hR, cnt2, et, se, w1, b1, w2, b2, gm, bt):
    half = pl.BlockSpec((BM, FH), lambda i: (i, 0))
    return pl.pallas_call(
        functools.partial(_mlp_body, relu_out),
        grid=(NB,),
        in_specs=[
            half, half, half, half,
            pl.BlockSpec((NC, BM, 16), lambda i: (0, i, 0)),
            pl.BlockSpec((16, F), lambda i: (0, 0)),
            pl.BlockSpec((1, F), lambda i: (0, 0)),
            pl.BlockSpec((F, H2), lambda i: (0, 0)),
            pl.BlockSpec((1, H2), lambda i: (0, 0)),
            pl.BlockSpec((H2, F), lambda i: (0, 0)),
            pl.BlockSpec((1, F), lambda i: (0, 0)),
            pl.BlockSpec((1, F), lambda i: (0, 0)),
            pl.BlockSpec((1, F), lambda i: (0, 0)),
        ],
        out_specs=(half, half),
        out_shape=(jax.ShapeDtypeStruct((NP, FH), F32),
                   jax.ShapeDtypeStruct((NP, FH), F32)),
    )(aL, aR, hL, hR, cnt2, et, se, w1, b1, w2, b2, gm, bt)


def _pool_body(hL, hR, b_ref, p1, pb1, p2, pb2, out_ref):
    i = pl.program_id(0)
    h = jnp.concatenate([hL[...], hR[...]], axis=1)
    t = jnp.maximum(jnp.dot(h, p1[...], preferred_element_type=F32) + pb1[...], 0.0)
    x = jnp.dot(t, p2[...], preferred_element_type=F32) + pb2[...]
    lane = lax.broadcasted_iota(I32, (1, PD), 1)
    x = x + (lane == PD - 1).astype(F32)
    b = b_ref[0]
    gio = lax.broadcasted_iota(I32, (G, BM), 0)
    ohT = (gio == b).astype(F32)
    contrib = jnp.dot(ohT, x, preferred_element_type=F32)

    @pl.when(i == 0)
    def _():
        out_ref[...] = contrib

    @pl.when(i > 0)
    def _():
        out_ref[...] = out_ref[...] + contrib

    @pl.when(i == NB - 1)
    def _():
        sums = out_ref[...]
        cnt = jnp.maximum(sums[:, PD - 1:PD], 1.0)
        out_ref[...] = sums / cnt


def _pool_call(hL, hR, batchr, p1, pb1, p2, pb2):
    half = pl.BlockSpec((BM, FH), lambda i: (i, 0))
    return pl.pallas_call(
        _pool_body,
        grid=(NB,),
        in_specs=[
            half, half,
            pl.BlockSpec((1, 1, BM), lambda i: (i, 0, 0)),
            pl.BlockSpec((F, F), lambda i: (0, 0)),
            pl.BlockSpec((1, F), lambda i: (0, 0)),
            pl.BlockSpec((F, PD), lambda i: (0, 0)),
            pl.BlockSpec((1, PD), lambda i: (0, 0)),
        ],
        out_specs=pl.BlockSpec((G, PD), lambda i: (0, 0)),
        out_shape=jax.ShapeDtypeStruct((G, PD), F32),
    )(hL, hR, batchr, p1, pb1, p2, pb2)


def kernel(x0, edge_index0, edge_attr, batch, atom_emb1, atom_emb2,
           edge_emb1, edge_emb2, W1, b1, W2, b2, gamma, beta, P1, pb1, P2, pb2):
    inv_std = 1.0 / jnp.sqrt(jnp.asarray(1.0 + 1e-5, F32))

    def padf(a, rows, cols):
        return jnp.zeros((rows, cols), F32).at[:a.shape[0], :a.shape[1]].set(a)

    t9 = (atom_emb1[:3][:, None, :] + atom_emb2[None, :3, :]).reshape(9, EMB)
    tab16 = padf(t9, 16, F)
    e9 = (edge_emb1[:, :3][:, :, None, :]
          + edge_emb2[:, None, :3, :]).reshape(NLAYER, 9, EMB)
    etabs = [padf(e9[l], 16, F) for l in range(NLAYER)]
    selfs = [padf((edge_emb1[l, 4] + edge_emb2[l, 0])[None, :], 1, F)
             for l in range(NLAYER)]
    W1p = [padf(W1[l], F, H2) for l in range(NLAYER)]
    b1p = [padf(b1[l][None, :], 1, H2) for l in range(NLAYER)]
    W2p = [padf(W2[l], H2, F) for l in range(NLAYER)]
    b2p = [padf(b2[l][None, :], 1, F) for l in range(NLAYER)]
    gmp = [padf((gamma[l] * inv_std)[None, :], 1, F) for l in range(NLAYER)]
    btp = [padf(beta[l][None, :], 1, F) for l in range(NLAYER)]
    P1p = padf(P1, F, F)
    pb1p = padf(pb1[None, :], 1, F)
    P2p = padf(P2, F, PD)
    pb2p = padf(pb2[None, :], 1, PD)

    x0r = jnp.zeros((NP, 2), I32).at[:N].set(x0.astype(I32)).reshape(NB, BM, 2)
    batchr = jnp.full((NP,), -1, I32).at[:N].set(batch.astype(I32)).reshape(NB, 1, BM)
    srca = edge_index0[0].astype(I32)
    dsta = edge_index0[1].astype(I32)
    ea = edge_attr.astype(I32)
    pk2 = (dsta << 16) | (ea[:, 0] << 8) | ea[:, 1]
    z16 = jnp.zeros((NPT, 16), F32)
    zrow = jnp.zeros((NPTS, FH), F32)

    hL, hR = _embed_call(x0r, tab16)
    cnt2 = _counts_call(pk2, z16)

    for l in range(NLAYER):
        aggL, aggR = _spmm_call(hL, hR, srca, dsta, zrow)
        hL, hR = _mlp_call(l < NLAYER - 1, aggL, aggR, hL, hR, cnt2,
                           etabs[l], selfs[l], W1p[l], b1p[l], W2p[l], b2p[l],
                           gmp[l], btp[l])

    pooled = _pool_call(hL, hR, batchr, P1p, pb1p, P2p, pb2p)
    return pooled[:, :150].reshape(-1)
